# baseline (speedup 1.0000x reference)
import functools

import jax
import jax.numpy as jnp
from jax import lax
from jax.experimental import pallas as pl
from jax.experimental.pallas import tpu as pltpu
from jax.experimental.pallas import tpu_sc as plsc

N = 10000
E = 320000
D = 128
EPS = 1e-7

NS = 16
EPT = E // NS
B = 40
NBLK = EPT // B
NPAD = 10240
ROWS = NPAD // NS
ZR = 8


def _sc_layer_body(x_hbm, p_hbm, src_hbm, dst_hbm, nude_hbm,
                   gidx, sidx, xrows, prows, orows, zbuf, acc,
                   isem0, isem1, isem2, isem3,
                   jsem0, jsem1,
                   gsem0, gsem1, psem0, psem1):
    c = lax.axis_index("c")
    s = lax.axis_index("s")
    isem = (isem0, isem1, isem2, isem3)
    jsem = (jsem0, jsem1)
    gsem = (gsem0, gsem1)
    psem = (psem0, psem1)

    def zrow(i, carry):
        for v in range(D // 16):
            zbuf[i, pl.ds(16 * v, 16)] = jnp.zeros((16,), jnp.float32)
        return carry

    lax.fori_loop(0, ZR, zrow, 0)

    def zcopy(r, carry):
        pltpu.sync_copy(zbuf, acc.at[pl.ds(s * ROWS + r * ZR, ZR)])
        return carry

    lax.fori_loop(0, ROWS // ZR, zcopy, 0)
    plsc.subcore_barrier()

    mul = jnp.where(c == 0, 1.0, 0.0).astype(jnp.float32)
    omm = 1.0 - mul

    pbase = s * EPT

    def stage_gidx(b, q):
        pltpu.async_copy(src_hbm.at[s, b], gidx.at[q], isem[q])

    def stage_sidx(sb, q):
        pltpu.async_copy(dst_hbm.at[s, sb], sidx.at[q], jsem[q])

    def start_loads(b, q, kd):
        pltpu.async_copy(x_hbm.at[gidx.at[q]], xrows.at[kd], gsem[kd])
        pltpu.async_copy(p_hbm.at[pl.ds(pbase + b * B, B), :], prows.at[kd],
                         psem[kd])

    def drain_gather(q, kd):
        pltpu.make_async_copy(x_hbm.at[gidx.at[q]], xrows.at[kd],
                              gsem[kd]).wait()

    def drain_prows(kd):
        pltpu.make_async_copy(p_hbm.at[pl.ds(0, B), :], prows.at[kd],
                              psem[kd]).wait()

    def drain_gidx(q):
        pltpu.make_async_copy(src_hbm.at[0, 0], gidx.at[q], isem[q]).wait()

    def drain_sidx(q):
        pltpu.make_async_copy(dst_hbm.at[0, 0], sidx.at[q], jsem[q]).wait()

    for q in range(4):
        stage_gidx(q, q)
    stage_sidx(0, 0)
    for kd in range(2):
        drain_gidx(kd)
        start_loads(kd, kd, kd)

    def quad(i, carry):
        for k in range(4):
            b = 4 * i + k
            kd = k % 2
            sb = 2 * i + (k // 2)

            drain_gather(k, kd)
            drain_prows(kd)

            @pl.when(b + 4 < NBLK)
            def _():
                stage_gidx(b + 4, k)

            if k == 0:
                stage_sidx(sb + 1, 1)
            if k == 2:
                @pl.when(sb + 1 < NBLK // 2)
                def _():
                    stage_sidx(sb + 1, 0)

            def edge(j, ecarry):
                for v in range(D // 16):
                    sl = pl.ds(16 * v, 16)
                    m = jnp.maximum(xrows[kd, j, sl] + prows[kd, j, sl], EPS)
                    p = jnp.exp(m)
                    orows[kd * B + j, sl] = p * (m * mul + omm)
                return ecarry

            lax.fori_loop(0, B, edge, 0)
            if k % 2 == 1:
                drain_sidx(k // 2)
                pltpu.sync_copy(orows, acc.at[sidx.at[k // 2]], add=True)

            @pl.when(b + 2 < NBLK)
            def _():
                drain_gidx((k + 2) % 4)
                start_loads(b + 2, (k + 2) % 4, kd)
        return carry

    lax.fori_loop(0, NBLK // 4, quad, 0)
    plsc.subcore_barrier()

    pltpu.sync_copy(acc.at[pl.ds(s * ROWS, ROWS)],
                    nude_hbm.at[c, pl.ds(s * ROWS, ROWS)])


_sc_layer = functools.partial(
    pl.kernel,
    out_type=jax.ShapeDtypeStruct((2, NPAD, D), jnp.float32),
    mesh=plsc.VectorSubcoreMesh(core_axis_name="c", subcore_axis_name="s"),
    scratch_types=[
        pltpu.VMEM((4, B), jnp.int32),
        pltpu.VMEM((2, 2 * B), jnp.int32),
        pltpu.VMEM((2, B, D), jnp.float32),
        pltpu.VMEM((2, B, D), jnp.float32),
        pltpu.VMEM((2 * B, D), jnp.float32),
        pltpu.VMEM((ZR, D), jnp.float32),
        pltpu.VMEM_SHARED((NPAD, D), jnp.float32),
    ] + [pltpu.SemaphoreType.DMA] * 10,
)(_sc_layer_body)


EB = 1280


def _proj_body(ea_ref, we1_ref, be1_ref, we2_ref, be2_ref, p1_ref, p2_ref):
    ea = ea_ref[...]
    p1_ref[...] = (jnp.dot(ea, we1_ref[...], preferred_element_type=jnp.float32)
                   + be1_ref[...] + EPS)
    p2_ref[...] = (jnp.dot(ea, we2_ref[...], preferred_element_type=jnp.float32)
                   + be2_ref[...] + EPS)


def _proj(ea, We1, be1, We2, be2):
    return pl.pallas_call(
        _proj_body,
        grid=(E // EB,),
        in_specs=[
            pl.BlockSpec((EB, 4), lambda i: (i, 0)),
            pl.BlockSpec((4, D), lambda i: (0, 0)),
            pl.BlockSpec((1, D), lambda i: (0, 0)),
            pl.BlockSpec((4, D), lambda i: (0, 0)),
            pl.BlockSpec((1, D), lambda i: (0, 0)),
        ],
        out_specs=[pl.BlockSpec((EB, D), lambda i: (i, 0)),
                   pl.BlockSpec((EB, D), lambda i: (i, 0))],
        out_shape=[jax.ShapeDtypeStruct((E, D), jnp.float32),
                   jax.ShapeDtypeStruct((E, D), jnp.float32)],
    )(ea, We1, be1.reshape(1, D), We2, be2.reshape(1, D))


RB = 1000


def _combine(x, numer, denom, W1, b1, W2, b2, leaky):
    H2 = W1.shape[1]

    def body(x_ref, nu_ref, de_ref, w1_ref, b1_ref, w2_ref, b2_ref, o_ref):
        xb = x_ref[...]
        sm = jnp.maximum(xb, 0.0) + EPS
        ps = jnp.exp(sm)
        aggr = (nu_ref[...] + ps * sm) / (de_ref[...] + ps)
        h = xb + aggr
        hh = (jnp.dot(h, w1_ref[...], preferred_element_type=jnp.float32)
              + b1_ref[...])
        hh = jnp.maximum(hh, 0.0)
        o = (jnp.dot(hh, w2_ref[...], preferred_element_type=jnp.float32)
             + b2_ref[...])
        if leaky:
            o = jnp.where(o > 0, o, 0.01 * o)
        o_ref[...] = o

    return pl.pallas_call(
        body,
        grid=(N // RB,),
        in_specs=[
            pl.BlockSpec((RB, D), lambda i: (i, 0)),
            pl.BlockSpec((RB, D), lambda i: (i, 0)),
            pl.BlockSpec((RB, D), lambda i: (i, 0)),
            pl.BlockSpec((D, H2), lambda i: (0, 0)),
            pl.BlockSpec((1, H2), lambda i: (0, 0)),
            pl.BlockSpec((H2, D), lambda i: (0, 0)),
            pl.BlockSpec((1, D), lambda i: (0, 0)),
        ],
        out_specs=pl.BlockSpec((RB, D), lambda i: (i, 0)),
        out_shape=jax.ShapeDtypeStruct((N, D), jnp.float32),
    )(x, numer, denom, W1, b1.reshape(1, H2), W2, b2.reshape(1, D))


def kernel(x, edge_index, edge_attr, We1, be1, W11, b11, W12, b12,
           We2, be2, W21, b21, W22, b22):
    src = edge_index[0].reshape(NS, NBLK, B)
    dst = edge_index[1].reshape(NS, NBLK // 2, 2 * B)
    p1, p2 = _proj(edge_attr, We1, be1, We2, be2)
    nude1 = _sc_layer(x, p1, src, dst)
    h1 = _combine(x, nude1[0], nude1[1], W11, b11, W12, b12, leaky=True)
    nude2 = _sc_layer(h1, p2, src, dst)
    out = _combine(h1, nude2[0], nude2[1], W21, b21, W22, b22, leaky=False)
    return out


# speedup vs baseline: 1.0669x; 1.0669x over previous
import functools

import jax
import jax.numpy as jnp
from jax import lax
from jax.experimental import pallas as pl
from jax.experimental.pallas import tpu as pltpu
from jax.experimental.pallas import tpu_sc as plsc

N = 10000
E = 320000
D = 128
EPS = 1e-7

NS = 16
EPT = E // NS
B = 40
NBLK = EPT // B
NPAD = 10240
ROWS = NPAD // NS
ZR = 8


def _sc_layer_body(x_hbm, p_hbm, src_hbm, dst_hbm, nude_hbm,
                   gidx, sidx, xrows, prows, orows, zbuf, acc,
                   isem0, isem1, isem2, isem3,
                   jsem0, jsem1, jsem2, jsem3,
                   gsem0, gsem1, psem0, psem1):
    c = lax.axis_index("c")
    s = lax.axis_index("s")
    isem = (isem0, isem1, isem2, isem3)
    jsem = (jsem0, jsem1, jsem2, jsem3)
    gsem = (gsem0, gsem1)
    psem = (psem0, psem1)

    def zrow(i, carry):
        for v in range(D // 16):
            zbuf[i, pl.ds(16 * v, 16)] = jnp.zeros((16,), jnp.float32)
        return carry

    lax.fori_loop(0, ZR, zrow, 0)

    def zcopy(r, carry):
        pltpu.sync_copy(zbuf, acc.at[pl.ds(s * ROWS + r * ZR, ZR)])
        return carry

    lax.fori_loop(0, ROWS // ZR, zcopy, 0)
    plsc.subcore_barrier()

    pbase = s * EPT

    def stage_gidx(b, q):
        pltpu.async_copy(src_hbm.at[s, b], gidx.at[q], isem[q])

    def stage_sidx(b, q):
        pltpu.async_copy(dst_hbm.at[s, b], sidx.at[q], jsem[q])

    def start_loads(b, q, kd):
        pltpu.async_copy(x_hbm.at[gidx.at[q]], xrows.at[kd], gsem[kd])
        pltpu.async_copy(p_hbm.at[pl.ds(pbase + b * B, B), :], prows.at[kd],
                         psem[kd])

    def drain_gather(q, kd):
        pltpu.make_async_copy(x_hbm.at[gidx.at[q]], xrows.at[kd],
                              gsem[kd]).wait()

    def drain_prows(kd):
        pltpu.make_async_copy(p_hbm.at[pl.ds(0, B), :], prows.at[kd],
                              psem[kd]).wait()

    def drain_gidx(q):
        pltpu.make_async_copy(src_hbm.at[0, 0], gidx.at[q], isem[q]).wait()

    def drain_sidx(q):
        pltpu.make_async_copy(dst_hbm.at[0, 0], sidx.at[q], jsem[q]).wait()

    for q in range(4):
        stage_gidx(q, q)
    for kd in range(2):
        stage_sidx(kd, kd)
        drain_gidx(kd)
        start_loads(kd, kd, kd)

    def quad(i, carry):
        for k in range(4):
            b = 4 * i + k
            kd = k % 2

            drain_gather(k, kd)
            drain_prows(kd)

            @pl.when(b + 4 < NBLK)
            def _():
                stage_gidx(b + 4, k)

            def edge_nu(j, ecarry):
                for v in range(D // 16):
                    sl = pl.ds(16 * v, 16)
                    m = jnp.maximum(xrows[kd, j, sl] + prows[kd, j, sl], EPS)
                    orows[kd, j, sl] = jnp.exp(m) * m
                return ecarry

            def edge_de(j, ecarry):
                for v in range(D // 16):
                    sl = pl.ds(16 * v, 16)
                    m = jnp.maximum(xrows[kd, j, sl] + prows[kd, j, sl], EPS)
                    orows[kd, j, sl] = jnp.exp(m)
                return ecarry

            @pl.when(c == 0)
            def _():
                lax.fori_loop(0, B, edge_nu, 0)

            @pl.when(c != 0)
            def _():
                lax.fori_loop(0, B, edge_de, 0)

            drain_sidx(k)
            pltpu.sync_copy(orows.at[kd], acc.at[sidx.at[k]], add=True)

            @pl.when(b + 2 < NBLK)
            def _():
                stage_sidx(b + 2, (k + 2) % 4)
                drain_gidx((k + 2) % 4)
                start_loads(b + 2, (k + 2) % 4, kd)
        return carry

    lax.fori_loop(0, NBLK // 4, quad, 0)
    plsc.subcore_barrier()

    pltpu.sync_copy(acc.at[pl.ds(s * ROWS, ROWS)],
                    nude_hbm.at[c, pl.ds(s * ROWS, ROWS)])


_sc_layer = functools.partial(
    pl.kernel,
    out_type=jax.ShapeDtypeStruct((2, NPAD, D), jnp.float32),
    mesh=plsc.VectorSubcoreMesh(core_axis_name="c", subcore_axis_name="s"),
    scratch_types=[
        pltpu.VMEM((4, B), jnp.int32),
        pltpu.VMEM((4, B), jnp.int32),
        pltpu.VMEM((2, B, D), jnp.float32),
        pltpu.VMEM((2, B, D), jnp.float32),
        pltpu.VMEM((2, B, D), jnp.float32),
        pltpu.VMEM((ZR, D), jnp.float32),
        pltpu.VMEM_SHARED((NPAD, D), jnp.float32),
    ] + [pltpu.SemaphoreType.DMA] * 12,
)(_sc_layer_body)


EB = 1280


def _proj_body(ea_ref, we1_ref, be1_ref, we2_ref, be2_ref, p1_ref, p2_ref):
    ea = ea_ref[...]
    p1_ref[...] = (jnp.dot(ea, we1_ref[...], preferred_element_type=jnp.float32)
                   + be1_ref[...] + EPS)
    p2_ref[...] = (jnp.dot(ea, we2_ref[...], preferred_element_type=jnp.float32)
                   + be2_ref[...] + EPS)


def _proj(ea, We1, be1, We2, be2):
    return pl.pallas_call(
        _proj_body,
        grid=(E // EB,),
        in_specs=[
            pl.BlockSpec((EB, 4), lambda i: (i, 0)),
            pl.BlockSpec((4, D), lambda i: (0, 0)),
            pl.BlockSpec((1, D), lambda i: (0, 0)),
            pl.BlockSpec((4, D), lambda i: (0, 0)),
            pl.BlockSpec((1, D), lambda i: (0, 0)),
        ],
        out_specs=[pl.BlockSpec((EB, D), lambda i: (i, 0)),
                   pl.BlockSpec((EB, D), lambda i: (i, 0))],
        out_shape=[jax.ShapeDtypeStruct((E, D), jnp.float32),
                   jax.ShapeDtypeStruct((E, D), jnp.float32)],
    )(ea, We1, be1.reshape(1, D), We2, be2.reshape(1, D))


RB = 1000


def _combine(x, numer, denom, W1, b1, W2, b2, leaky):
    H2 = W1.shape[1]

    def body(x_ref, nu_ref, de_ref, w1_ref, b1_ref, w2_ref, b2_ref, o_ref):
        xb = x_ref[...]
        sm = jnp.maximum(xb, 0.0) + EPS
        ps = jnp.exp(sm)
        aggr = (nu_ref[...] + ps * sm) / (de_ref[...] + ps)
        h = xb + aggr
        hh = (jnp.dot(h, w1_ref[...], preferred_element_type=jnp.float32)
              + b1_ref[...])
        hh = jnp.maximum(hh, 0.0)
        o = (jnp.dot(hh, w2_ref[...], preferred_element_type=jnp.float32)
             + b2_ref[...])
        if leaky:
            o = jnp.where(o > 0, o, 0.01 * o)
        o_ref[...] = o

    return pl.pallas_call(
        body,
        grid=(N // RB,),
        in_specs=[
            pl.BlockSpec((RB, D), lambda i: (i, 0)),
            pl.BlockSpec((RB, D), lambda i: (i, 0)),
            pl.BlockSpec((RB, D), lambda i: (i, 0)),
            pl.BlockSpec((D, H2), lambda i: (0, 0)),
            pl.BlockSpec((1, H2), lambda i: (0, 0)),
            pl.BlockSpec((H2, D), lambda i: (0, 0)),
            pl.BlockSpec((1, D), lambda i: (0, 0)),
        ],
        out_specs=pl.BlockSpec((RB, D), lambda i: (i, 0)),
        out_shape=jax.ShapeDtypeStruct((N, D), jnp.float32),
    )(x, numer, denom, W1, b1.reshape(1, H2), W2, b2.reshape(1, D))


def kernel(x, edge_index, edge_attr, We1, be1, W11, b11, W12, b12,
           We2, be2, W21, b21, W22, b22):
    src = edge_index[0].reshape(NS, NBLK, B)
    dst = edge_index[1].reshape(NS, NBLK, B)
    p1, p2 = _proj(edge_attr, We1, be1, We2, be2)
    nude1 = _sc_layer(x, p1, src, dst)
    h1 = _combine(x, nude1[0], nude1[1], W11, b11, W12, b12, leaky=True)
    nude2 = _sc_layer(h1, p2, src, dst)
    out = _combine(h1, nude2[0], nude2[1], W21, b21, W22, b22, leaky=False)
    return out


# speedup vs baseline: 1.1468x; 1.0749x over previous
import functools

import jax
import jax.numpy as jnp
from jax import lax
from jax.experimental import pallas as pl
from jax.experimental.pallas import tpu as pltpu
from jax.experimental.pallas import tpu_sc as plsc

N = 10000
E = 320000
D = 128
EPS = 1e-7

NS = 16
EPT = E // NS
B = 40
NBLK = EPT // B
NPAD = 10240
ROWS = NPAD // NS
ZR = 8


def _sc_layer_body(x_hbm, p_hbm, src_hbm, dst_hbm, nude_hbm,
                   gidx, sidx, xrows, prows, orows, zbuf, acc,
                   isem0, isem1, isem2, isem3,
                   jsem0, jsem1, jsem2, jsem3,
                   gsem0, gsem1, psem0, psem1):
    c = lax.axis_index("c")
    s = lax.axis_index("s")
    isem = (isem0, isem1, isem2, isem3)
    jsem = (jsem0, jsem1, jsem2, jsem3)
    gsem = (gsem0, gsem1)
    psem = (psem0, psem1)

    def zrow(i, carry):
        for v in range(D // 16):
            zbuf[i, pl.ds(16 * v, 16)] = jnp.zeros((16,), jnp.float32)
        return carry

    lax.fori_loop(0, ZR, zrow, 0)

    def zcopy(r, carry):
        pltpu.sync_copy(zbuf, acc.at[pl.ds(s * ROWS + r * ZR, ZR)])
        return carry

    lax.fori_loop(0, ROWS // ZR, zcopy, 0)
    plsc.subcore_barrier()

    pbase = s * EPT

    def stage_gidx(b, q):
        pltpu.async_copy(src_hbm.at[s, b], gidx.at[q], isem[q])

    def stage_sidx(b, q):
        pltpu.async_copy(dst_hbm.at[s, b], sidx.at[q], jsem[q])

    def start_loads(b, q, kd):
        pltpu.async_copy(x_hbm.at[gidx.at[q]], xrows.at[kd], gsem[kd])
        pltpu.async_copy(p_hbm.at[pl.ds(pbase + b * B, B), :], prows.at[kd],
                         psem[kd])

    def drain_gather(q, kd):
        pltpu.make_async_copy(x_hbm.at[gidx.at[q]], xrows.at[kd],
                              gsem[kd]).wait()

    def drain_prows(kd):
        pltpu.make_async_copy(p_hbm.at[pl.ds(0, B), :], prows.at[kd],
                              psem[kd]).wait()

    def drain_gidx(q):
        pltpu.make_async_copy(src_hbm.at[0, 0], gidx.at[q], isem[q]).wait()

    def drain_sidx(q):
        pltpu.make_async_copy(dst_hbm.at[0, 0], sidx.at[q], jsem[q]).wait()

    for q in range(4):
        stage_gidx(q, q)
    for kd in range(2):
        stage_sidx(kd, kd)
        drain_gidx(kd)
        start_loads(kd, kd, kd)

    def quad(i, carry):
        for k in range(4):
            b = 4 * i + k
            kd = k % 2

            drain_gather(k, kd)
            drain_prows(kd)

            @pl.when(b + 4 < NBLK)
            def _():
                stage_gidx(b + 4, k)

            def edge_nu(j, ecarry):
                for v in range(D // 16):
                    sl = pl.ds(16 * v, 16)
                    m = jnp.maximum(xrows[kd, j, sl] + prows[kd, j, sl], EPS)
                    orows[kd, j, sl] = jnp.exp(m) * m
                return ecarry

            def edge_de(j, ecarry):
                for v in range(D // 16):
                    sl = pl.ds(16 * v, 16)
                    m = jnp.maximum(xrows[kd, j, sl] + prows[kd, j, sl], EPS)
                    orows[kd, j, sl] = jnp.exp(m)
                return ecarry

            @pl.when(c == 0)
            def _():
                lax.fori_loop(0, B, edge_nu, 0)

            @pl.when(c != 0)
            def _():
                lax.fori_loop(0, B, edge_de, 0)

            drain_sidx(k)
            pltpu.sync_copy(orows.at[kd], acc.at[sidx.at[k]], add=True)

            @pl.when(b + 2 < NBLK)
            def _():
                stage_sidx(b + 2, (k + 2) % 4)
                drain_gidx((k + 2) % 4)
                start_loads(b + 2, (k + 2) % 4, kd)
        return carry

    lax.fori_loop(0, NBLK // 4, quad, 0)
    plsc.subcore_barrier()

    pltpu.sync_copy(acc.at[pl.ds(s * ROWS, ROWS)],
                    nude_hbm.at[c, pl.ds(s * ROWS, ROWS)])


_sc_layer = functools.partial(
    pl.kernel,
    out_type=jax.ShapeDtypeStruct((2, NPAD, D), jnp.float32),
    mesh=plsc.VectorSubcoreMesh(core_axis_name="c", subcore_axis_name="s"),
    scratch_types=[
        pltpu.VMEM((4, B), jnp.int32),
        pltpu.VMEM((4, B), jnp.int32),
        pltpu.VMEM((2, B, D), jnp.float32),
        pltpu.VMEM((2, B, D), jnp.float32),
        pltpu.VMEM((2, B, D), jnp.float32),
        pltpu.VMEM((ZR, D), jnp.float32),
        pltpu.VMEM_SHARED((NPAD, D), jnp.float32),
    ] + [pltpu.SemaphoreType.DMA] * 12,
)(_sc_layer_body)


EB = 3200


def _proj_body(ea_ref, we1_ref, be1_ref, we2_ref, be2_ref, p1_ref, p2_ref):
    ea = ea_ref[...]
    p1_ref[...] = (jnp.dot(ea, we1_ref[...], preferred_element_type=jnp.float32)
                   + be1_ref[...] + EPS)
    p2_ref[...] = (jnp.dot(ea, we2_ref[...], preferred_element_type=jnp.float32)
                   + be2_ref[...] + EPS)


def _proj(ea, We1, be1, We2, be2):
    return pl.pallas_call(
        _proj_body,
        grid=(E // EB,),
        in_specs=[
            pl.BlockSpec((EB, 4), lambda i: (i, 0)),
            pl.BlockSpec((4, D), lambda i: (0, 0)),
            pl.BlockSpec((1, D), lambda i: (0, 0)),
            pl.BlockSpec((4, D), lambda i: (0, 0)),
            pl.BlockSpec((1, D), lambda i: (0, 0)),
        ],
        out_specs=[pl.BlockSpec((EB, D), lambda i: (i, 0)),
                   pl.BlockSpec((EB, D), lambda i: (i, 0))],
        out_shape=[jax.ShapeDtypeStruct((E, D), jnp.float32),
                   jax.ShapeDtypeStruct((E, D), jnp.float32)],
    )(ea, We1, be1.reshape(1, D), We2, be2.reshape(1, D))


RB = 2000


def _combine(x, numer, denom, W1, b1, W2, b2, leaky):
    H2 = W1.shape[1]

    def body(x_ref, nu_ref, de_ref, w1_ref, b1_ref, w2_ref, b2_ref, o_ref):
        xb = x_ref[...]
        sm = jnp.maximum(xb, 0.0) + EPS
        ps = jnp.exp(sm)
        aggr = (nu_ref[...] + ps * sm) / (de_ref[...] + ps)
        h = xb + aggr
        hh = (jnp.dot(h, w1_ref[...], preferred_element_type=jnp.float32)
              + b1_ref[...])
        hh = jnp.maximum(hh, 0.0)
        o = (jnp.dot(hh, w2_ref[...], preferred_element_type=jnp.float32)
             + b2_ref[...])
        if leaky:
            o = jnp.where(o > 0, o, 0.01 * o)
        o_ref[...] = o

    return pl.pallas_call(
        body,
        grid=(N // RB,),
        in_specs=[
            pl.BlockSpec((RB, D), lambda i: (i, 0)),
            pl.BlockSpec((RB, D), lambda i: (i, 0)),
            pl.BlockSpec((RB, D), lambda i: (i, 0)),
            pl.BlockSpec((D, H2), lambda i: (0, 0)),
            pl.BlockSpec((1, H2), lambda i: (0, 0)),
            pl.BlockSpec((H2, D), lambda i: (0, 0)),
            pl.BlockSpec((1, D), lambda i: (0, 0)),
        ],
        out_specs=pl.BlockSpec((RB, D), lambda i: (i, 0)),
        out_shape=jax.ShapeDtypeStruct((N, D), jnp.float32),
    )(x, numer, denom, W1, b1.reshape(1, H2), W2, b2.reshape(1, D))


def kernel(x, edge_index, edge_attr, We1, be1, W11, b11, W12, b12,
           We2, be2, W21, b21, W22, b22):
    src = edge_index[0].reshape(NS, NBLK, B)
    dst = edge_index[1].reshape(NS, NBLK, B)
    p1, p2 = _proj(edge_attr, We1, be1, We2, be2)
    nude1 = _sc_layer(x, p1, src, dst)
    h1 = _combine(x, nude1[0], nude1[1], W11, b11, W12, b12, leaky=True)
    nude2 = _sc_layer(h1, p2, src, dst)
    out = _combine(h1, nude2[0], nude2[1], W21, b21, W22, b22, leaky=False)
    return out
